# baseline (speedup 1.0000x reference)
import functools

import jax
import jax.numpy as jnp
from jax import lax
from jax.experimental import pallas as pl
from jax.experimental.pallas import tpu as pltpu
from jax.experimental.pallas import tpu_sc as plsc

_HUMAN_CONF = 0.7
_THRESH = 0.25
_IOU_THR = 0.5
_BIG = 2**30
_CAP = 1024
_B, _N, _C = 4, 20000, 80
_R, _L = 8, _N // 8
_WIN = 2504
_WBUF = 2520
_STEPS = (_L + 15) // 16


def _compact_sc(hl_hbm, pb_hbm, sw_hbm, sh_hbm,
                cur_hbm, x1_hbm, y1_hbm, x2_hbm, y2_hbm, idx_hbm, flag_hbm,
                l0_v, l1_v, cx_v, cy_v, w_v, h_v, sw_v, sh_v,
                bc_v, bx1_v, by1_v, bx2_v, by2_v, bi_v, fl_v):
    wid = lax.axis_index("s") * 2 + lax.axis_index("c")
    b = wid // _R
    w = wid % _R
    rbase = w * _L
    astart = (rbase // 8) * 8
    off = rbase - astart

    pltpu.sync_copy(hl_hbm.at[pl.ds((b * 2 + 0) * _N + astart, _WIN)], l0_v.at[pl.ds(0, _WIN)])
    pltpu.sync_copy(hl_hbm.at[pl.ds((b * 2 + 1) * _N + astart, _WIN)], l1_v.at[pl.ds(0, _WIN)])
    pltpu.sync_copy(pb_hbm.at[pl.ds((b * 4 + 0) * _N + astart, _WIN)], cx_v.at[pl.ds(0, _WIN)])
    pltpu.sync_copy(pb_hbm.at[pl.ds((b * 4 + 1) * _N + astart, _WIN)], cy_v.at[pl.ds(0, _WIN)])
    pltpu.sync_copy(pb_hbm.at[pl.ds((b * 4 + 2) * _N + astart, _WIN)], w_v.at[pl.ds(0, _WIN)])
    pltpu.sync_copy(pb_hbm.at[pl.ds((b * 4 + 3) * _N + astart, _WIN)], h_v.at[pl.ds(0, _WIN)])
    pltpu.sync_copy(sw_hbm, sw_v)
    pltpu.sync_copy(sh_hbm, sh_v)
    sw = sw_v[...]
    sh = sh_v[...]

    zf = jnp.zeros((16,), jnp.float32)
    sent = jnp.full((16,), -4.0, jnp.float32)
    bigv = jnp.full((16,), _BIG, jnp.int32)

    def prefill(i, _):
        bc_v[pl.ds(i * 16, 16)] = sent
        bx1_v[pl.ds(i * 16, 16)] = zf
        by1_v[pl.ds(i * 16, 16)] = zf
        bx2_v[pl.ds(i * 16, 16)] = zf
        by2_v[pl.ds(i * 16, 16)] = zf
        bi_v[pl.ds(i * 16, 16)] = bigv
        return 0

    lax.fori_loop(0, (_CAP + 16) // 16, prefill, 0)

    def step(i, cnt):
        start = off + i * 16
        rel = i * 16 + lax.iota(jnp.int32, 16)
        valid = rel < _L
        l0 = l0_v[pl.ds(start, 16)]
        l1 = l1_v[pl.ds(start, 16)]
        cx = cx_v[pl.ds(start, 16)]
        cy = cy_v[pl.ds(start, 16)]
        ww = w_v[pl.ds(start, 16)]
        hh = h_v[pl.ds(start, 16)]
        m = jnp.maximum(l0, l1)
        e0 = jnp.exp(l0 - m)
        e1 = jnp.exp(l1 - m)
        ssum = e0 + e1
        p = jnp.maximum(e0 / ssum, e1 / ssum)
        cand = (l0 >= l1) & (p >= _HUMAN_CONF) & valid
        x1 = (cx - 0.5 * ww) * sw
        y1 = (cy - 0.5 * hh) * sh
        x2 = (cx + 0.5 * ww) * sw
        y2 = (cy + 0.5 * hh) * sh
        oidx = jnp.broadcast_to(rbase, (16,)) + rel
        npop = plsc.all_reduce_population_count(cand)[0]

        @pl.when(cnt + npop <= _CAP)
        def _():
            plsc.store_compressed(bc_v.at[pl.ds(cnt, 16)], p, mask=cand)
            plsc.store_compressed(bx1_v.at[pl.ds(cnt, 16)], x1, mask=cand)
            plsc.store_compressed(by1_v.at[pl.ds(cnt, 16)], y1, mask=cand)
            plsc.store_compressed(bx2_v.at[pl.ds(cnt, 16)], x2, mask=cand)
            plsc.store_compressed(by2_v.at[pl.ds(cnt, 16)], y2, mask=cand)
            plsc.store_compressed(bi_v.at[pl.ds(cnt, 16)], oidx, mask=cand)

        return cnt + npop

    cnt = lax.fori_loop(0, _STEPS, step, jnp.int32(0))

    ovf = jnp.where(cnt > _CAP, jnp.int32(1), jnp.int32(0))
    fl_v[...] = jnp.broadcast_to(ovf, (16,))
    dst = wid * _CAP
    pltpu.sync_copy(fl_v.at[pl.ds(0, 8)], flag_hbm.at[pl.ds(wid * 8, 8)])
    pltpu.sync_copy(bc_v.at[pl.ds(0, _CAP)], cur_hbm.at[pl.ds(dst, _CAP)])
    pltpu.sync_copy(bx1_v.at[pl.ds(0, _CAP)], x1_hbm.at[pl.ds(dst, _CAP)])
    pltpu.sync_copy(by1_v.at[pl.ds(0, _CAP)], y1_hbm.at[pl.ds(dst, _CAP)])
    pltpu.sync_copy(bx2_v.at[pl.ds(0, _CAP)], x2_hbm.at[pl.ds(dst, _CAP)])
    pltpu.sync_copy(by2_v.at[pl.ds(0, _CAP)], y2_hbm.at[pl.ds(dst, _CAP)])
    pltpu.sync_copy(bi_v.at[pl.ds(0, _CAP)], idx_hbm.at[pl.ds(dst, _CAP)])


def _greedy_nms(cur0, x1, y1, x2, y2, area, jidx):
    neg = jnp.float32(float("-inf"))

    def _max3(v):
        return jnp.max(jnp.max(v, axis=2, keepdims=True), axis=1, keepdims=True)

    supp = jnp.float32(-4.0)
    mb0 = _max3(cur0)

    def _cond(st):
        _, mb = st
        return jnp.max(mb) > 0.0

    def _body(st):
        cur, mb = st
        selj = jnp.where(cur == mb, jidx, _BIG)
        ib = jnp.min(jnp.min(selj, axis=2, keepdims=True), axis=1, keepdims=True)
        ib = jnp.where(mb > 0.0, ib, -1)
        is_sel = jidx == ib

        def pick(v):
            return _max3(jnp.where(is_sel, v, neg))

        x1i = pick(x1)
        y1i = pick(y1)
        x2i = pick(x2)
        y2i = pick(y2)
        ai = (x2i - x1i) * (y2i - y1i)
        xx1 = jnp.maximum(x1i, x1)
        yy1 = jnp.maximum(y1i, y1)
        xx2 = jnp.minimum(x2i, x2)
        yy2 = jnp.minimum(y2i, y2)
        inter = jnp.maximum(0.0, xx2 - xx1) * jnp.maximum(0.0, yy2 - yy1)
        iou = inter / (ai + area - inter + 1e-12)
        cur = jnp.where(is_sel, -cur, jnp.where(iou > _IOU_THR, supp, cur))
        return cur, _max3(cur)

    def _body4(st):
        return _body(_body(_body(_body(st))))

    cur, _ = lax.while_loop(_cond, _body4, (cur0, mb0))
    return cur


def _nms_body(scale_ref, flag_ref, cur_ref, x1_ref, y1_ref, x2_ref, y2_ref,
              idx_ref, hl_ref, pb_ref, keep_ref, jout_ref):
    overflow = jnp.max(flag_ref[...]) > 0

    @pl.when(jnp.logical_not(overflow))
    def _fast():
        cur0 = cur_ref[...]
        x1 = x1_ref[...]
        y1 = y1_ref[...]
        x2 = x2_ref[...]
        y2 = y2_ref[...]
        area = (x2 - x1) * (y2 - y1)
        jidx = idx_ref[...]
        cur = _greedy_nms(cur0, x1, y1, x2, y2, area, jidx)
        keep_ref[...] = jnp.zeros((_B, _R, _L), jnp.float32)
        jout_ref[...] = jnp.full((_B, _R, _L), _BIG, jnp.int32)
        keep_ref[:, :, 0:_CAP] = jnp.where(cur > -2.0, 1.0, 0.0)
        jout_ref[:, :, 0:_CAP] = jidx

    @pl.when(overflow)
    def _full():
        l0 = hl_ref[:, 0]
        l1 = hl_ref[:, 1]
        m = jnp.maximum(l0, l1)
        e0 = jnp.exp(l0 - m)
        e1 = jnp.exp(l1 - m)
        s = e0 + e1
        score = jnp.maximum(e0 / s, e1 / s)
        cand = (l0 >= l1) & (score >= _HUMAN_CONF)
        sw = scale_ref[0]
        sh = scale_ref[1]
        cx = pb_ref[:, 0]
        cy = pb_ref[:, 1]
        w = pb_ref[:, 2]
        h = pb_ref[:, 3]
        x1 = (cx - 0.5 * w) * sw
        y1 = (cy - 0.5 * h) * sh
        x2 = (cx + 0.5 * w) * sw
        y2 = (cy + 0.5 * h) * sh
        area = (x2 - x1) * (y2 - y1)
        jrow = lax.broadcasted_iota(jnp.int32, (_B, _R, _L), 1)
        jcol = lax.broadcasted_iota(jnp.int32, (_B, _R, _L), 2)
        jidx = jrow * _L + jcol
        cur0 = jnp.where(cand, score, jnp.float32(-4.0))
        cur = _greedy_nms(cur0, x1, y1, x2, y2, area, jidx)
        keep_ref[...] = jnp.where(cur > -2.0, 1.0, 0.0)
        jout_ref[...] = jidx


def _scatter_sc(keep_hbm, jout_hbm, out_hbm, kv, jv, dest_v):
    wid = lax.axis_index("s") * 2 + lax.axis_index("c")

    @pl.when(wid < _B)
    def _():
        b = wid
        pltpu.sync_copy(keep_hbm.at[pl.ds(b * _N, _N)], kv)
        pltpu.sync_copy(jout_hbm.at[pl.ds(b * _N, _N)], jv)
        zf = jnp.zeros((16,), jnp.float32)

        def zstep(i, _):
            dest_v[pl.ds(i * 16, 16)] = zf
            return 0

        lax.fori_loop(0, _N // 16, zstep, 0)
        ones = jnp.ones((16,), jnp.float32)

        def sstep(i, _):
            k = kv[pl.ds(i * 16, 16)]
            j = jv[pl.ds(i * 16, 16)]
            msk = k > 0.5
            plsc.store_scatter(dest_v, [j], ones, mask=msk)
            return 0

        lax.fori_loop(0, _N // 16, sstep, 0)
        pltpu.sync_copy(dest_v, out_hbm.at[pl.ds(b * _N, _N)])


def _mask_body(scale_ref, keep_ref, logit_ref, pb_ref, out_ref, bxi_ref):
    k = keep_ref[0]
    x = logit_ref[0]
    sel = (k > 0.0) & (x >= _THRESH)
    out_ref[0] = jnp.where(sel, (x + 1.0) * 0.5, 0.0)

    sw = scale_ref[0]
    sh = scale_ref[1]
    cx = pb_ref[0, 0:1]
    cy = pb_ref[0, 1:2]
    w = pb_ref[0, 2:3]
    h = pb_ref[0, 3:4]
    kb = k > 0.0
    zero = jnp.float32(0.0)
    bxi_ref[0, 0:1] = jnp.where(kb, (cx - 0.5 * w) * sw, zero).astype(jnp.int32)
    bxi_ref[0, 1:2] = jnp.where(kb, (cy - 0.5 * h) * sh, zero).astype(jnp.int32)
    bxi_ref[0, 2:3] = jnp.where(kb, (cx + 0.5 * w) * sw, zero).astype(jnp.int32)
    bxi_ref[0, 3:4] = jnp.where(kb, (cy + 0.5 * h) * sh, zero).astype(jnp.int32)


def kernel(human_logits, pred_logits, pred_boxes, img_h, img_w):
    B, N, C = pred_logits.shape
    scale = jnp.stack([img_w, img_h, img_w, img_h]).astype(jnp.float32)
    sw16 = jnp.broadcast_to(scale[0], (16,))
    sh16 = jnp.broadcast_to(scale[1], (16,))

    hl_t = human_logits.transpose(0, 2, 1)
    pb_t = pred_boxes.transpose(0, 2, 1)

    mesh = plsc.VectorSubcoreMesh(core_axis_name="c", subcore_axis_name="s")
    f32 = jnp.float32
    nflat = B * _R * _CAP
    compact = pl.kernel(
        _compact_sc,
        out_type=[
            jax.ShapeDtypeStruct((nflat,), f32),
            jax.ShapeDtypeStruct((nflat,), f32),
            jax.ShapeDtypeStruct((nflat,), f32),
            jax.ShapeDtypeStruct((nflat,), f32),
            jax.ShapeDtypeStruct((nflat,), f32),
            jax.ShapeDtypeStruct((nflat,), jnp.int32),
            jax.ShapeDtypeStruct((B * _R * 8,), jnp.int32),
        ],
        mesh=mesh,
        compiler_params=pltpu.CompilerParams(needs_layout_passes=False),
        scratch_types=[
            pltpu.VMEM((_WBUF,), f32),
            pltpu.VMEM((_WBUF,), f32),
            pltpu.VMEM((_WBUF,), f32),
            pltpu.VMEM((_WBUF,), f32),
            pltpu.VMEM((_WBUF,), f32),
            pltpu.VMEM((_WBUF,), f32),
            pltpu.VMEM((16,), f32),
            pltpu.VMEM((16,), f32),
            pltpu.VMEM((_CAP + 16,), f32),
            pltpu.VMEM((_CAP + 16,), f32),
            pltpu.VMEM((_CAP + 16,), f32),
            pltpu.VMEM((_CAP + 16,), f32),
            pltpu.VMEM((_CAP + 16,), f32),
            pltpu.VMEM((_CAP + 16,), jnp.int32),
            pltpu.VMEM((16,), jnp.int32),
        ],
    )
    curc, x1c, y1c, x2c, y2c, idxc, flags = compact(
        hl_t.reshape(B * 2 * N), pb_t.reshape(B * 4 * N), sw16, sh16
    )
    curc = curc.reshape(B, _R, _CAP)
    x1c = x1c.reshape(B, _R, _CAP)
    y1c = y1c.reshape(B, _R, _CAP)
    x2c = x2c.reshape(B, _R, _CAP)
    y2c = y2c.reshape(B, _R, _CAP)
    idxc = idxc.reshape(B, _R, _CAP)
    flags = flags.reshape(B, _R, 8)

    hl_r = hl_t.reshape(B, 2, _R, _L)
    pb_r = pb_t.reshape(B, 4, _R, _L)

    keep_plane, jout_plane = pl.pallas_call(
        _nms_body,
        in_specs=[
            pl.BlockSpec(memory_space=pltpu.SMEM),
            pl.BlockSpec(memory_space=pltpu.VMEM),
            pl.BlockSpec(memory_space=pltpu.VMEM),
            pl.BlockSpec(memory_space=pltpu.VMEM),
            pl.BlockSpec(memory_space=pltpu.VMEM),
            pl.BlockSpec(memory_space=pltpu.VMEM),
            pl.BlockSpec(memory_space=pltpu.VMEM),
            pl.BlockSpec(memory_space=pltpu.VMEM),
            pl.BlockSpec(memory_space=pltpu.VMEM),
            pl.BlockSpec(memory_space=pltpu.VMEM),
        ],
        out_specs=[
            pl.BlockSpec(memory_space=pltpu.VMEM),
            pl.BlockSpec(memory_space=pltpu.VMEM),
        ],
        out_shape=[
            jax.ShapeDtypeStruct((B, _R, _L), f32),
            jax.ShapeDtypeStruct((B, _R, _L), jnp.int32),
        ],
    )(scale, flags, curc, x1c, y1c, x2c, y2c, idxc, hl_r, pb_r)

    scatter = pl.kernel(
        _scatter_sc,
        out_type=[jax.ShapeDtypeStruct((B * N,), f32)],
        mesh=mesh,
        compiler_params=pltpu.CompilerParams(needs_layout_passes=False),
        scratch_types=[
            pltpu.VMEM((N,), f32),
            pltpu.VMEM((N,), jnp.int32),
            pltpu.VMEM((N,), f32),
        ],
    )
    (keep_orig,) = scatter(keep_plane.reshape(B * N), jout_plane.reshape(B * N))
    keep_orig = keep_orig.reshape(B, 1, N)

    logits_t = pred_logits.transpose(0, 2, 1)

    scores_t, bxi = pl.pallas_call(
        _mask_body,
        grid=(B,),
        in_specs=[
            pl.BlockSpec(memory_space=pltpu.SMEM),
            pl.BlockSpec((1, 1, N), lambda b: (b, 0, 0)),
            pl.BlockSpec((1, C, N), lambda b: (b, 0, 0)),
            pl.BlockSpec((1, 4, N), lambda b: (b, 0, 0)),
        ],
        out_specs=[
            pl.BlockSpec((1, C, N), lambda b: (b, 0, 0)),
            pl.BlockSpec((1, 4, N), lambda b: (b, 0, 0)),
        ],
        out_shape=[
            jax.ShapeDtypeStruct((B, C, N), f32),
            jax.ShapeDtypeStruct((B, 4, N), jnp.int32),
        ],
    )(scale, keep_orig, logits_t, pb_t)

    return scores_t.transpose(0, 2, 1), bxi.transpose(0, 2, 1)


# speedup vs baseline: 1.0341x; 1.0341x over previous
import functools

import jax
import jax.numpy as jnp
from jax import lax
from jax.experimental import pallas as pl
from jax.experimental.pallas import tpu as pltpu
from jax.experimental.pallas import tpu_sc as plsc

_HUMAN_CONF = 0.7
_THRESH = 0.25
_IOU_THR = 0.5
_BIG = 2**30
_CAP = 1024
_B, _N, _C = 4, 20000, 80
_R, _L = 8, _N // 8
_WIN = 2504
_WBUF = 2520
_STEPS = (_L + 15) // 16


def _compact_sc(hl_hbm, pb_hbm, sw_hbm, sh_hbm,
                cur_hbm, x1_hbm, y1_hbm, x2_hbm, y2_hbm, idx_hbm, flag_hbm,
                l0_v, l1_v, cx_v, cy_v, w_v, h_v, sw_v, sh_v,
                bc_v, bx1_v, by1_v, bx2_v, by2_v, bi_v, fl_v):
    wid = lax.axis_index("s") * 2 + lax.axis_index("c")
    b = wid // _R
    w = wid % _R
    rbase = w * _L
    astart = (rbase // 8) * 8
    off = rbase - astart

    pltpu.sync_copy(hl_hbm.at[pl.ds((b * 2 + 0) * _N + astart, _WIN)], l0_v.at[pl.ds(0, _WIN)])
    pltpu.sync_copy(hl_hbm.at[pl.ds((b * 2 + 1) * _N + astart, _WIN)], l1_v.at[pl.ds(0, _WIN)])
    pltpu.sync_copy(pb_hbm.at[pl.ds((b * 4 + 0) * _N + astart, _WIN)], cx_v.at[pl.ds(0, _WIN)])
    pltpu.sync_copy(pb_hbm.at[pl.ds((b * 4 + 1) * _N + astart, _WIN)], cy_v.at[pl.ds(0, _WIN)])
    pltpu.sync_copy(pb_hbm.at[pl.ds((b * 4 + 2) * _N + astart, _WIN)], w_v.at[pl.ds(0, _WIN)])
    pltpu.sync_copy(pb_hbm.at[pl.ds((b * 4 + 3) * _N + astart, _WIN)], h_v.at[pl.ds(0, _WIN)])
    pltpu.sync_copy(sw_hbm, sw_v)
    pltpu.sync_copy(sh_hbm, sh_v)
    sw = sw_v[...]
    sh = sh_v[...]

    zf = jnp.zeros((16,), jnp.float32)
    sent = jnp.full((16,), -4.0, jnp.float32)
    bigv = jnp.full((16,), _BIG, jnp.int32)

    def prefill(i, _):
        bc_v[pl.ds(i * 16, 16)] = sent
        bx1_v[pl.ds(i * 16, 16)] = zf
        by1_v[pl.ds(i * 16, 16)] = zf
        bx2_v[pl.ds(i * 16, 16)] = zf
        by2_v[pl.ds(i * 16, 16)] = zf
        bi_v[pl.ds(i * 16, 16)] = bigv
        return 0

    lax.fori_loop(0, (_CAP + 16) // 16, prefill, 0)

    def step(i, cnt):
        start = off + i * 16
        rel = i * 16 + lax.iota(jnp.int32, 16)
        valid = rel < _L
        l0 = l0_v[pl.ds(start, 16)]
        l1 = l1_v[pl.ds(start, 16)]
        cx = cx_v[pl.ds(start, 16)]
        cy = cy_v[pl.ds(start, 16)]
        ww = w_v[pl.ds(start, 16)]
        hh = h_v[pl.ds(start, 16)]
        m = jnp.maximum(l0, l1)
        e0 = jnp.exp(l0 - m)
        e1 = jnp.exp(l1 - m)
        ssum = e0 + e1
        p = jnp.maximum(e0 / ssum, e1 / ssum)
        cand = (l0 >= l1) & (p >= _HUMAN_CONF) & valid
        x1 = (cx - 0.5 * ww) * sw
        y1 = (cy - 0.5 * hh) * sh
        x2 = (cx + 0.5 * ww) * sw
        y2 = (cy + 0.5 * hh) * sh
        oidx = jnp.broadcast_to(rbase, (16,)) + rel
        npop = plsc.all_reduce_population_count(cand)[0]

        @pl.when(cnt + npop <= _CAP)
        def _():
            plsc.store_compressed(bc_v.at[pl.ds(cnt, 16)], p, mask=cand)
            plsc.store_compressed(bx1_v.at[pl.ds(cnt, 16)], x1, mask=cand)
            plsc.store_compressed(by1_v.at[pl.ds(cnt, 16)], y1, mask=cand)
            plsc.store_compressed(bx2_v.at[pl.ds(cnt, 16)], x2, mask=cand)
            plsc.store_compressed(by2_v.at[pl.ds(cnt, 16)], y2, mask=cand)
            plsc.store_compressed(bi_v.at[pl.ds(cnt, 16)], oidx, mask=cand)

        return cnt + npop

    cnt = lax.fori_loop(0, _STEPS, step, jnp.int32(0))

    ovf = jnp.where(cnt > _CAP, jnp.int32(1), jnp.int32(0))
    fl_v[...] = jnp.broadcast_to(ovf, (16,))
    dst = wid * _CAP
    pltpu.sync_copy(fl_v.at[pl.ds(0, 8)], flag_hbm.at[pl.ds(wid * 8, 8)])
    pltpu.sync_copy(bc_v.at[pl.ds(0, _CAP)], cur_hbm.at[pl.ds(dst, _CAP)])
    pltpu.sync_copy(bx1_v.at[pl.ds(0, _CAP)], x1_hbm.at[pl.ds(dst, _CAP)])
    pltpu.sync_copy(by1_v.at[pl.ds(0, _CAP)], y1_hbm.at[pl.ds(dst, _CAP)])
    pltpu.sync_copy(bx2_v.at[pl.ds(0, _CAP)], x2_hbm.at[pl.ds(dst, _CAP)])
    pltpu.sync_copy(by2_v.at[pl.ds(0, _CAP)], y2_hbm.at[pl.ds(dst, _CAP)])
    pltpu.sync_copy(bi_v.at[pl.ds(0, _CAP)], idx_hbm.at[pl.ds(dst, _CAP)])


def _greedy_nms(cur0, x1, y1, x2, y2, area, jidx):
    neg = jnp.float32(float("-inf"))

    def _max3(v):
        return jnp.max(jnp.max(v, axis=2, keepdims=True), axis=1, keepdims=True)

    supp = jnp.float32(-4.0)
    mb0 = _max3(cur0)

    def _cond(st):
        _, mb = st
        return jnp.max(mb) > 0.0

    def _body(st):
        cur, mb = st
        selj = jnp.where(cur == mb, jidx, _BIG)
        ib = jnp.min(jnp.min(selj, axis=2, keepdims=True), axis=1, keepdims=True)
        ib = jnp.where(mb > 0.0, ib, -1)
        is_sel = jidx == ib

        def pick(v):
            return _max3(jnp.where(is_sel, v, neg))

        x1i = pick(x1)
        y1i = pick(y1)
        x2i = pick(x2)
        y2i = pick(y2)
        ai = (x2i - x1i) * (y2i - y1i)
        xx1 = jnp.maximum(x1i, x1)
        yy1 = jnp.maximum(y1i, y1)
        xx2 = jnp.minimum(x2i, x2)
        yy2 = jnp.minimum(y2i, y2)
        inter = jnp.maximum(0.0, xx2 - xx1) * jnp.maximum(0.0, yy2 - yy1)
        iou = inter / (ai + area - inter + 1e-12)
        cur = jnp.where(is_sel, -cur, jnp.where(iou > _IOU_THR, supp, cur))
        return cur, _max3(cur)

    def _body8(st):
        for _ in range(8):
            st = _body(st)
        return st

    cur, _ = lax.while_loop(_cond, _body8, (cur0, mb0))
    return cur


def _nms_body(scale_ref, flag_ref, cur_ref, x1_ref, y1_ref, x2_ref, y2_ref,
              idx_ref, hl_ref, pb_ref, keep_ref, jout_ref):
    overflow = jnp.max(flag_ref[...]) > 0

    @pl.when(jnp.logical_not(overflow))
    def _fast():
        cur0 = cur_ref[...]
        x1 = x1_ref[...]
        y1 = y1_ref[...]
        x2 = x2_ref[...]
        y2 = y2_ref[...]
        area = (x2 - x1) * (y2 - y1)
        jidx = idx_ref[...]
        cur = _greedy_nms(cur0, x1, y1, x2, y2, area, jidx)
        keep_ref[...] = jnp.zeros((_B, _R, _L), jnp.float32)
        jout_ref[...] = jnp.full((_B, _R, _L), _BIG, jnp.int32)
        keep_ref[:, :, 0:_CAP] = jnp.where(cur > -2.0, 1.0, 0.0)
        jout_ref[:, :, 0:_CAP] = jidx

    @pl.when(overflow)
    def _full():
        l0 = hl_ref[:, 0]
        l1 = hl_ref[:, 1]
        m = jnp.maximum(l0, l1)
        e0 = jnp.exp(l0 - m)
        e1 = jnp.exp(l1 - m)
        s = e0 + e1
        score = jnp.maximum(e0 / s, e1 / s)
        cand = (l0 >= l1) & (score >= _HUMAN_CONF)
        sw = scale_ref[0]
        sh = scale_ref[1]
        cx = pb_ref[:, 0]
        cy = pb_ref[:, 1]
        w = pb_ref[:, 2]
        h = pb_ref[:, 3]
        x1 = (cx - 0.5 * w) * sw
        y1 = (cy - 0.5 * h) * sh
        x2 = (cx + 0.5 * w) * sw
        y2 = (cy + 0.5 * h) * sh
        area = (x2 - x1) * (y2 - y1)
        jrow = lax.broadcasted_iota(jnp.int32, (_B, _R, _L), 1)
        jcol = lax.broadcasted_iota(jnp.int32, (_B, _R, _L), 2)
        jidx = jrow * _L + jcol
        cur0 = jnp.where(cand, score, jnp.float32(-4.0))
        cur = _greedy_nms(cur0, x1, y1, x2, y2, area, jidx)
        keep_ref[...] = jnp.where(cur > -2.0, 1.0, 0.0)
        jout_ref[...] = jidx


def _scatter_sc(keep_hbm, jout_hbm, out_hbm, kv, jv, dest_v):
    wid = lax.axis_index("s") * 2 + lax.axis_index("c")

    @pl.when(wid < _B)
    def _():
        b = wid
        pltpu.sync_copy(keep_hbm.at[pl.ds(b * _N, _N)], kv)
        pltpu.sync_copy(jout_hbm.at[pl.ds(b * _N, _N)], jv)
        zf = jnp.zeros((16,), jnp.float32)

        def zstep(i, _):
            dest_v[pl.ds(i * 16, 16)] = zf
            return 0

        lax.fori_loop(0, _N // 16, zstep, 0)
        ones = jnp.ones((16,), jnp.float32)

        def sstep(i, _):
            k = kv[pl.ds(i * 16, 16)]
            j = jv[pl.ds(i * 16, 16)]
            msk = k > 0.5
            plsc.store_scatter(dest_v, [j], ones, mask=msk)
            return 0

        lax.fori_loop(0, _N // 16, sstep, 0)
        pltpu.sync_copy(dest_v, out_hbm.at[pl.ds(b * _N, _N)])


def _mask_body(scale_ref, keep_ref, logit_ref, pb_ref, out_ref, bxi_ref):
    k = keep_ref[0]
    x = logit_ref[0]
    sel = (k > 0.0) & (x >= _THRESH)
    out_ref[0] = jnp.where(sel, (x + 1.0) * 0.5, 0.0)

    sw = scale_ref[0]
    sh = scale_ref[1]
    cx = pb_ref[0, 0:1]
    cy = pb_ref[0, 1:2]
    w = pb_ref[0, 2:3]
    h = pb_ref[0, 3:4]
    kb = k > 0.0
    zero = jnp.float32(0.0)
    bxi_ref[0, 0:1] = jnp.where(kb, (cx - 0.5 * w) * sw, zero).astype(jnp.int32)
    bxi_ref[0, 1:2] = jnp.where(kb, (cy - 0.5 * h) * sh, zero).astype(jnp.int32)
    bxi_ref[0, 2:3] = jnp.where(kb, (cx + 0.5 * w) * sw, zero).astype(jnp.int32)
    bxi_ref[0, 3:4] = jnp.where(kb, (cy + 0.5 * h) * sh, zero).astype(jnp.int32)


def kernel(human_logits, pred_logits, pred_boxes, img_h, img_w):
    B, N, C = pred_logits.shape
    scale = jnp.stack([img_w, img_h, img_w, img_h]).astype(jnp.float32)
    sw16 = jnp.broadcast_to(scale[0], (16,))
    sh16 = jnp.broadcast_to(scale[1], (16,))

    hl_t = human_logits.transpose(0, 2, 1)
    pb_t = pred_boxes.transpose(0, 2, 1)

    mesh = plsc.VectorSubcoreMesh(core_axis_name="c", subcore_axis_name="s")
    f32 = jnp.float32
    nflat = B * _R * _CAP
    compact = pl.kernel(
        _compact_sc,
        out_type=[
            jax.ShapeDtypeStruct((nflat,), f32),
            jax.ShapeDtypeStruct((nflat,), f32),
            jax.ShapeDtypeStruct((nflat,), f32),
            jax.ShapeDtypeStruct((nflat,), f32),
            jax.ShapeDtypeStruct((nflat,), f32),
            jax.ShapeDtypeStruct((nflat,), jnp.int32),
            jax.ShapeDtypeStruct((B * _R * 8,), jnp.int32),
        ],
        mesh=mesh,
        compiler_params=pltpu.CompilerParams(needs_layout_passes=False),
        scratch_types=[
            pltpu.VMEM((_WBUF,), f32),
            pltpu.VMEM((_WBUF,), f32),
            pltpu.VMEM((_WBUF,), f32),
            pltpu.VMEM((_WBUF,), f32),
            pltpu.VMEM((_WBUF,), f32),
            pltpu.VMEM((_WBUF,), f32),
            pltpu.VMEM((16,), f32),
            pltpu.VMEM((16,), f32),
            pltpu.VMEM((_CAP + 16,), f32),
            pltpu.VMEM((_CAP + 16,), f32),
            pltpu.VMEM((_CAP + 16,), f32),
            pltpu.VMEM((_CAP + 16,), f32),
            pltpu.VMEM((_CAP + 16,), f32),
            pltpu.VMEM((_CAP + 16,), jnp.int32),
            pltpu.VMEM((16,), jnp.int32),
        ],
    )
    curc, x1c, y1c, x2c, y2c, idxc, flags = compact(
        hl_t.reshape(B * 2 * N), pb_t.reshape(B * 4 * N), sw16, sh16
    )
    curc = curc.reshape(B, _R, _CAP)
    x1c = x1c.reshape(B, _R, _CAP)
    y1c = y1c.reshape(B, _R, _CAP)
    x2c = x2c.reshape(B, _R, _CAP)
    y2c = y2c.reshape(B, _R, _CAP)
    idxc = idxc.reshape(B, _R, _CAP)
    flags = flags.reshape(B, _R, 8)

    hl_r = hl_t.reshape(B, 2, _R, _L)
    pb_r = pb_t.reshape(B, 4, _R, _L)

    keep_plane, jout_plane = pl.pallas_call(
        _nms_body,
        in_specs=[
            pl.BlockSpec(memory_space=pltpu.SMEM),
            pl.BlockSpec(memory_space=pltpu.VMEM),
            pl.BlockSpec(memory_space=pltpu.VMEM),
            pl.BlockSpec(memory_space=pltpu.VMEM),
            pl.BlockSpec(memory_space=pltpu.VMEM),
            pl.BlockSpec(memory_space=pltpu.VMEM),
            pl.BlockSpec(memory_space=pltpu.VMEM),
            pl.BlockSpec(memory_space=pltpu.VMEM),
            pl.BlockSpec(memory_space=pltpu.VMEM),
            pl.BlockSpec(memory_space=pltpu.VMEM),
        ],
        out_specs=[
            pl.BlockSpec(memory_space=pltpu.VMEM),
            pl.BlockSpec(memory_space=pltpu.VMEM),
        ],
        out_shape=[
            jax.ShapeDtypeStruct((B, _R, _L), f32),
            jax.ShapeDtypeStruct((B, _R, _L), jnp.int32),
        ],
    )(scale, flags, curc, x1c, y1c, x2c, y2c, idxc, hl_r, pb_r)

    scatter = pl.kernel(
        _scatter_sc,
        out_type=[jax.ShapeDtypeStruct((B * N,), f32)],
        mesh=mesh,
        compiler_params=pltpu.CompilerParams(needs_layout_passes=False),
        scratch_types=[
            pltpu.VMEM((N,), f32),
            pltpu.VMEM((N,), jnp.int32),
            pltpu.VMEM((N,), f32),
        ],
    )
    (keep_orig,) = scatter(keep_plane.reshape(B * N), jout_plane.reshape(B * N))
    keep_orig = keep_orig.reshape(B, 1, N)

    logits_t = pred_logits.transpose(0, 2, 1)

    scores_t, bxi = pl.pallas_call(
        _mask_body,
        grid=(B,),
        in_specs=[
            pl.BlockSpec(memory_space=pltpu.SMEM),
            pl.BlockSpec((1, 1, N), lambda b: (b, 0, 0)),
            pl.BlockSpec((1, C, N), lambda b: (b, 0, 0)),
            pl.BlockSpec((1, 4, N), lambda b: (b, 0, 0)),
        ],
        out_specs=[
            pl.BlockSpec((1, C, N), lambda b: (b, 0, 0)),
            pl.BlockSpec((1, 4, N), lambda b: (b, 0, 0)),
        ],
        out_shape=[
            jax.ShapeDtypeStruct((B, C, N), f32),
            jax.ShapeDtypeStruct((B, 4, N), jnp.int32),
        ],
    )(scale, keep_orig, logits_t, pb_t)

    return scores_t.transpose(0, 2, 1), bxi.transpose(0, 2, 1)


# speedup vs baseline: 1.0551x; 1.0203x over previous
import functools

import jax
import jax.numpy as jnp
from jax import lax
from jax.experimental import pallas as pl
from jax.experimental.pallas import tpu as pltpu
from jax.experimental.pallas import tpu_sc as plsc

_HUMAN_CONF = 0.7
_THRESH = 0.25
_IOU_THR = 0.5
_BIG = 2**30
_CAP = 1024
_B, _N, _C = 4, 20000, 80
_R, _L = 8, _N // 8
_WIN = 2504
_WBUF = 2520
_STEPS = (_L + 15) // 16


def _compact_sc(hl_hbm, pb_hbm, sw_hbm, sh_hbm,
                cur_hbm, x1_hbm, y1_hbm, x2_hbm, y2_hbm, idx_hbm, flag_hbm,
                l0_v, l1_v, cx_v, cy_v, w_v, h_v, sw_v, sh_v,
                bc_v, bx1_v, by1_v, bx2_v, by2_v, bi_v, fl_v):
    wid = lax.axis_index("s") * 2 + lax.axis_index("c")
    b = wid // _R
    w = wid % _R
    rbase = w * _L
    astart = (rbase // 8) * 8
    off = rbase - astart

    pltpu.sync_copy(hl_hbm.at[pl.ds((b * 2 + 0) * _N + astart, _WIN)], l0_v.at[pl.ds(0, _WIN)])
    pltpu.sync_copy(hl_hbm.at[pl.ds((b * 2 + 1) * _N + astart, _WIN)], l1_v.at[pl.ds(0, _WIN)])
    pltpu.sync_copy(pb_hbm.at[pl.ds((b * 4 + 0) * _N + astart, _WIN)], cx_v.at[pl.ds(0, _WIN)])
    pltpu.sync_copy(pb_hbm.at[pl.ds((b * 4 + 1) * _N + astart, _WIN)], cy_v.at[pl.ds(0, _WIN)])
    pltpu.sync_copy(pb_hbm.at[pl.ds((b * 4 + 2) * _N + astart, _WIN)], w_v.at[pl.ds(0, _WIN)])
    pltpu.sync_copy(pb_hbm.at[pl.ds((b * 4 + 3) * _N + astart, _WIN)], h_v.at[pl.ds(0, _WIN)])
    pltpu.sync_copy(sw_hbm, sw_v)
    pltpu.sync_copy(sh_hbm, sh_v)
    sw = sw_v[...]
    sh = sh_v[...]

    zf = jnp.zeros((16,), jnp.float32)
    sent = jnp.full((16,), -4.0, jnp.float32)
    bigv = jnp.full((16,), _BIG, jnp.int32)

    def prefill(i, _):
        bc_v[pl.ds(i * 16, 16)] = sent
        bx1_v[pl.ds(i * 16, 16)] = zf
        by1_v[pl.ds(i * 16, 16)] = zf
        bx2_v[pl.ds(i * 16, 16)] = zf
        by2_v[pl.ds(i * 16, 16)] = zf
        bi_v[pl.ds(i * 16, 16)] = bigv
        return 0

    lax.fori_loop(0, (_CAP + 16) // 16, prefill, 0)

    def step(i, cnt):
        start = off + i * 16
        rel = i * 16 + lax.iota(jnp.int32, 16)
        valid = rel < _L
        l0 = l0_v[pl.ds(start, 16)]
        l1 = l1_v[pl.ds(start, 16)]
        cx = cx_v[pl.ds(start, 16)]
        cy = cy_v[pl.ds(start, 16)]
        ww = w_v[pl.ds(start, 16)]
        hh = h_v[pl.ds(start, 16)]
        m = jnp.maximum(l0, l1)
        e0 = jnp.exp(l0 - m)
        e1 = jnp.exp(l1 - m)
        ssum = e0 + e1
        p = jnp.maximum(e0 / ssum, e1 / ssum)
        cand = (l0 >= l1) & (p >= _HUMAN_CONF) & valid
        x1 = (cx - 0.5 * ww) * sw
        y1 = (cy - 0.5 * hh) * sh
        x2 = (cx + 0.5 * ww) * sw
        y2 = (cy + 0.5 * hh) * sh
        oidx = jnp.broadcast_to(rbase, (16,)) + rel
        npop = plsc.all_reduce_population_count(cand)[0]

        @pl.when(cnt + npop <= _CAP)
        def _():
            plsc.store_compressed(bc_v.at[pl.ds(cnt, 16)], p, mask=cand)
            plsc.store_compressed(bx1_v.at[pl.ds(cnt, 16)], x1, mask=cand)
            plsc.store_compressed(by1_v.at[pl.ds(cnt, 16)], y1, mask=cand)
            plsc.store_compressed(bx2_v.at[pl.ds(cnt, 16)], x2, mask=cand)
            plsc.store_compressed(by2_v.at[pl.ds(cnt, 16)], y2, mask=cand)
            plsc.store_compressed(bi_v.at[pl.ds(cnt, 16)], oidx, mask=cand)

        return cnt + npop

    cnt = lax.fori_loop(0, _STEPS, step, jnp.int32(0))

    ovf = jnp.where(cnt > _CAP, jnp.int32(1), jnp.int32(0))
    fl_v[...] = jnp.broadcast_to(ovf, (16,))
    dst = wid * _CAP
    pltpu.sync_copy(fl_v.at[pl.ds(0, 8)], flag_hbm.at[pl.ds(wid * 8, 8)])
    pltpu.sync_copy(bc_v.at[pl.ds(0, _CAP)], cur_hbm.at[pl.ds(dst, _CAP)])
    pltpu.sync_copy(bx1_v.at[pl.ds(0, _CAP)], x1_hbm.at[pl.ds(dst, _CAP)])
    pltpu.sync_copy(by1_v.at[pl.ds(0, _CAP)], y1_hbm.at[pl.ds(dst, _CAP)])
    pltpu.sync_copy(bx2_v.at[pl.ds(0, _CAP)], x2_hbm.at[pl.ds(dst, _CAP)])
    pltpu.sync_copy(by2_v.at[pl.ds(0, _CAP)], y2_hbm.at[pl.ds(dst, _CAP)])
    pltpu.sync_copy(bi_v.at[pl.ds(0, _CAP)], idx_hbm.at[pl.ds(dst, _CAP)])


def _greedy_nms(cur0, x1, y1, x2, y2, area, jidx):
    neg = jnp.float32(float("-inf"))

    def _max3(v):
        return jnp.max(jnp.max(v, axis=2, keepdims=True), axis=1, keepdims=True)

    supp = jnp.float32(-4.0)
    mb0 = _max3(cur0)

    def _cond(st):
        _, mb = st
        return jnp.max(mb) > 0.0

    def _body(st):
        cur, mb = st
        selj = jnp.where(cur == mb, jidx, _BIG)
        ib = jnp.min(jnp.min(selj, axis=2, keepdims=True), axis=1, keepdims=True)
        ib = jnp.where(mb > 0.0, ib, -1)
        is_sel = jidx == ib

        def pick(v):
            return _max3(jnp.where(is_sel, v, neg))

        x1i = pick(x1)
        y1i = pick(y1)
        x2i = pick(x2)
        y2i = pick(y2)
        ai = (x2i - x1i) * (y2i - y1i)
        xx1 = jnp.maximum(x1i, x1)
        yy1 = jnp.maximum(y1i, y1)
        xx2 = jnp.minimum(x2i, x2)
        yy2 = jnp.minimum(y2i, y2)
        inter = jnp.maximum(0.0, xx2 - xx1) * jnp.maximum(0.0, yy2 - yy1)
        iou = inter / (ai + area - inter + 1e-12)
        cur = jnp.where(is_sel, -cur, jnp.where(iou > _IOU_THR, supp, cur))
        return cur, _max3(cur)

    def _bodyk(st):
        for _ in range(16):
            st = _body(st)
        return st

    cur, _ = lax.while_loop(_cond, _bodyk, (cur0, mb0))
    return cur


def _nms_body(scale_ref, flag_ref, cur_ref, x1_ref, y1_ref, x2_ref, y2_ref,
              idx_ref, hl_ref, pb_ref, keep_ref, jout_ref):
    overflow = jnp.max(flag_ref[...]) > 0

    @pl.when(jnp.logical_not(overflow))
    def _fast():
        cur0 = cur_ref[...]
        x1 = x1_ref[...]
        y1 = y1_ref[...]
        x2 = x2_ref[...]
        y2 = y2_ref[...]
        area = (x2 - x1) * (y2 - y1)
        jidx = idx_ref[...]
        cur = _greedy_nms(cur0, x1, y1, x2, y2, area, jidx)
        keep_ref[...] = jnp.zeros((_B, _R, _L), jnp.float32)
        jout_ref[...] = jnp.full((_B, _R, _L), _BIG, jnp.int32)
        keep_ref[:, :, 0:_CAP] = jnp.where(cur > -2.0, 1.0, 0.0)
        jout_ref[:, :, 0:_CAP] = jidx

    @pl.when(overflow)
    def _full():
        l0 = hl_ref[:, 0]
        l1 = hl_ref[:, 1]
        m = jnp.maximum(l0, l1)
        e0 = jnp.exp(l0 - m)
        e1 = jnp.exp(l1 - m)
        s = e0 + e1
        score = jnp.maximum(e0 / s, e1 / s)
        cand = (l0 >= l1) & (score >= _HUMAN_CONF)
        sw = scale_ref[0]
        sh = scale_ref[1]
        cx = pb_ref[:, 0]
        cy = pb_ref[:, 1]
        w = pb_ref[:, 2]
        h = pb_ref[:, 3]
        x1 = (cx - 0.5 * w) * sw
        y1 = (cy - 0.5 * h) * sh
        x2 = (cx + 0.5 * w) * sw
        y2 = (cy + 0.5 * h) * sh
        area = (x2 - x1) * (y2 - y1)
        jrow = lax.broadcasted_iota(jnp.int32, (_B, _R, _L), 1)
        jcol = lax.broadcasted_iota(jnp.int32, (_B, _R, _L), 2)
        jidx = jrow * _L + jcol
        cur0 = jnp.where(cand, score, jnp.float32(-4.0))
        cur = _greedy_nms(cur0, x1, y1, x2, y2, area, jidx)
        keep_ref[...] = jnp.where(cur > -2.0, 1.0, 0.0)
        jout_ref[...] = jidx


def _scatter_sc(keep_hbm, jout_hbm, out_hbm, kv, jv, dest_v):
    wid = lax.axis_index("s") * 2 + lax.axis_index("c")

    @pl.when(wid < _B)
    def _():
        b = wid
        pltpu.sync_copy(keep_hbm.at[pl.ds(b * _N, _N)], kv)
        pltpu.sync_copy(jout_hbm.at[pl.ds(b * _N, _N)], jv)
        zf = jnp.zeros((16,), jnp.float32)

        def zstep(i, _):
            dest_v[pl.ds(i * 16, 16)] = zf
            return 0

        lax.fori_loop(0, _N // 16, zstep, 0)
        ones = jnp.ones((16,), jnp.float32)

        def sstep(i, _):
            k = kv[pl.ds(i * 16, 16)]
            j = jv[pl.ds(i * 16, 16)]
            msk = k > 0.5
            plsc.store_scatter(dest_v, [j], ones, mask=msk)
            return 0

        lax.fori_loop(0, _N // 16, sstep, 0)
        pltpu.sync_copy(dest_v, out_hbm.at[pl.ds(b * _N, _N)])


def _mask_body(scale_ref, keep_ref, logit_ref, pb_ref, out_ref, bxi_ref):
    k = keep_ref[0]
    x = logit_ref[0]
    sel = (k > 0.0) & (x >= _THRESH)
    out_ref[0] = jnp.where(sel, (x + 1.0) * 0.5, 0.0)

    sw = scale_ref[0]
    sh = scale_ref[1]
    cx = pb_ref[0, 0:1]
    cy = pb_ref[0, 1:2]
    w = pb_ref[0, 2:3]
    h = pb_ref[0, 3:4]
    kb = k > 0.0
    zero = jnp.float32(0.0)
    bxi_ref[0, 0:1] = jnp.where(kb, (cx - 0.5 * w) * sw, zero).astype(jnp.int32)
    bxi_ref[0, 1:2] = jnp.where(kb, (cy - 0.5 * h) * sh, zero).astype(jnp.int32)
    bxi_ref[0, 2:3] = jnp.where(kb, (cx + 0.5 * w) * sw, zero).astype(jnp.int32)
    bxi_ref[0, 3:4] = jnp.where(kb, (cy + 0.5 * h) * sh, zero).astype(jnp.int32)


def kernel(human_logits, pred_logits, pred_boxes, img_h, img_w):
    B, N, C = pred_logits.shape
    scale = jnp.stack([img_w, img_h, img_w, img_h]).astype(jnp.float32)
    sw16 = jnp.broadcast_to(scale[0], (16,))
    sh16 = jnp.broadcast_to(scale[1], (16,))

    hl_t = human_logits.transpose(0, 2, 1)
    pb_t = pred_boxes.transpose(0, 2, 1)

    mesh = plsc.VectorSubcoreMesh(core_axis_name="c", subcore_axis_name="s")
    f32 = jnp.float32
    nflat = B * _R * _CAP
    compact = pl.kernel(
        _compact_sc,
        out_type=[
            jax.ShapeDtypeStruct((nflat,), f32),
            jax.ShapeDtypeStruct((nflat,), f32),
            jax.ShapeDtypeStruct((nflat,), f32),
            jax.ShapeDtypeStruct((nflat,), f32),
            jax.ShapeDtypeStruct((nflat,), f32),
            jax.ShapeDtypeStruct((nflat,), jnp.int32),
            jax.ShapeDtypeStruct((B * _R * 8,), jnp.int32),
        ],
        mesh=mesh,
        compiler_params=pltpu.CompilerParams(needs_layout_passes=False),
        scratch_types=[
            pltpu.VMEM((_WBUF,), f32),
            pltpu.VMEM((_WBUF,), f32),
            pltpu.VMEM((_WBUF,), f32),
            pltpu.VMEM((_WBUF,), f32),
            pltpu.VMEM((_WBUF,), f32),
            pltpu.VMEM((_WBUF,), f32),
            pltpu.VMEM((16,), f32),
            pltpu.VMEM((16,), f32),
            pltpu.VMEM((_CAP + 16,), f32),
            pltpu.VMEM((_CAP + 16,), f32),
            pltpu.VMEM((_CAP + 16,), f32),
            pltpu.VMEM((_CAP + 16,), f32),
            pltpu.VMEM((_CAP + 16,), f32),
            pltpu.VMEM((_CAP + 16,), jnp.int32),
            pltpu.VMEM((16,), jnp.int32),
        ],
    )
    curc, x1c, y1c, x2c, y2c, idxc, flags = compact(
        hl_t.reshape(B * 2 * N), pb_t.reshape(B * 4 * N), sw16, sh16
    )
    curc = curc.reshape(B, _R, _CAP)
    x1c = x1c.reshape(B, _R, _CAP)
    y1c = y1c.reshape(B, _R, _CAP)
    x2c = x2c.reshape(B, _R, _CAP)
    y2c = y2c.reshape(B, _R, _CAP)
    idxc = idxc.reshape(B, _R, _CAP)
    flags = flags.reshape(B, _R, 8)

    hl_r = hl_t.reshape(B, 2, _R, _L)
    pb_r = pb_t.reshape(B, 4, _R, _L)

    keep_plane, jout_plane = pl.pallas_call(
        _nms_body,
        in_specs=[
            pl.BlockSpec(memory_space=pltpu.SMEM),
            pl.BlockSpec(memory_space=pltpu.VMEM),
            pl.BlockSpec(memory_space=pltpu.VMEM),
            pl.BlockSpec(memory_space=pltpu.VMEM),
            pl.BlockSpec(memory_space=pltpu.VMEM),
            pl.BlockSpec(memory_space=pltpu.VMEM),
            pl.BlockSpec(memory_space=pltpu.VMEM),
            pl.BlockSpec(memory_space=pltpu.VMEM),
            pl.BlockSpec(memory_space=pltpu.VMEM),
            pl.BlockSpec(memory_space=pltpu.VMEM),
        ],
        out_specs=[
            pl.BlockSpec(memory_space=pltpu.VMEM),
            pl.BlockSpec(memory_space=pltpu.VMEM),
        ],
        out_shape=[
            jax.ShapeDtypeStruct((B, _R, _L), f32),
            jax.ShapeDtypeStruct((B, _R, _L), jnp.int32),
        ],
    )(scale, flags, curc, x1c, y1c, x2c, y2c, idxc, hl_r, pb_r)

    scatter = pl.kernel(
        _scatter_sc,
        out_type=[jax.ShapeDtypeStruct((B * N,), f32)],
        mesh=mesh,
        compiler_params=pltpu.CompilerParams(needs_layout_passes=False),
        scratch_types=[
            pltpu.VMEM((N,), f32),
            pltpu.VMEM((N,), jnp.int32),
            pltpu.VMEM((N,), f32),
        ],
    )
    (keep_orig,) = scatter(keep_plane.reshape(B * N), jout_plane.reshape(B * N))
    keep_orig = keep_orig.reshape(B, 1, N)

    logits_t = pred_logits.transpose(0, 2, 1)

    scores_t, bxi = pl.pallas_call(
        _mask_body,
        grid=(B,),
        in_specs=[
            pl.BlockSpec(memory_space=pltpu.SMEM),
            pl.BlockSpec((1, 1, N), lambda b: (b, 0, 0)),
            pl.BlockSpec((1, C, N), lambda b: (b, 0, 0)),
            pl.BlockSpec((1, 4, N), lambda b: (b, 0, 0)),
        ],
        out_specs=[
            pl.BlockSpec((1, C, N), lambda b: (b, 0, 0)),
            pl.BlockSpec((1, 4, N), lambda b: (b, 0, 0)),
        ],
        out_shape=[
            jax.ShapeDtypeStruct((B, C, N), f32),
            jax.ShapeDtypeStruct((B, 4, N), jnp.int32),
        ],
    )(scale, keep_orig, logits_t, pb_t)

    return scores_t.transpose(0, 2, 1), bxi.transpose(0, 2, 1)
